# baseline (speedup 1.0000x reference)
import jax
import jax.numpy as jnp
from jax.experimental import pallas as pl
import jax.experimental.pallas.tpu as pltpu

_B = 16
_C = 768
_P = 512
_HW = 1024
_KI = 8
_W = 4


def _concat_pos_kernel(x_hbm, row_ref, col_ref, o_hbm, cbuf, posb, in_sems, out_sems, pos_sem):
    colb = jnp.broadcast_to(col_ref[...][None, :, :], (32, 32, 256)).reshape(_HW, 256)
    rowb = jnp.broadcast_to(row_ref[...][:, None, :], (32, 32, 256)).reshape(_HW, 256)
    posb[:, :256] = colb
    posb[:, 256:] = rowb

    pos_copies = [
        pltpu.make_async_copy(posb, o_hbm.at[i, :, _C:], pos_sem)
        for i in range(_B)
    ]
    for cp in pos_copies:
        cp.start()

    def in_copy(i):
        return pltpu.make_async_copy(x_hbm.at[i], cbuf.at[i % _KI], in_sems.at[i % _KI])

    out_copies = [
        pltpu.make_async_copy(cbuf.at[i % _KI], o_hbm.at[i, :, 0:_C], out_sems.at[i % _KI])
        for i in range(_B)
    ]

    for i in range(_KI):
        in_copy(i).start()
    for i in range(_B):
        in_copy(i).wait()
        out_copies[i].start()
        j = i - _W
        if j >= 0 and j + _KI < _B:
            out_copies[j].wait()
            in_copy(j + _KI).start()
    for i in range(_B - _KI, _B):
        out_copies[i].wait()
    for cp in pos_copies:
        cp.wait()


def kernel(x, row_embed, col_embed):
    b, c, h, w = x.shape
    xt = x.transpose(0, 2, 3, 1).reshape(b, h * w, c)
    out = pl.pallas_call(
        _concat_pos_kernel,
        in_specs=[
            pl.BlockSpec(memory_space=pl.ANY),
            pl.BlockSpec(memory_space=pltpu.MemorySpace.VMEM),
            pl.BlockSpec(memory_space=pltpu.MemorySpace.VMEM),
        ],
        out_specs=pl.BlockSpec(memory_space=pl.ANY),
        out_shape=jax.ShapeDtypeStruct((b, h * w, c + _P), x.dtype),
        scratch_shapes=[
            pltpu.VMEM((_KI, h * w, c), x.dtype),
            pltpu.VMEM((h * w, _P), x.dtype),
            pltpu.SemaphoreType.DMA((_KI,)),
            pltpu.SemaphoreType.DMA((_KI,)),
            pltpu.SemaphoreType.DMA,
        ],
    )(xt, row_embed, col_embed)
    return out.reshape(b, h, w, c + _P).transpose(0, 3, 1, 2)


# speedup vs baseline: 1.0137x; 1.0137x over previous
import jax
import jax.numpy as jnp
from jax.experimental import pallas as pl
import jax.experimental.pallas.tpu as pltpu

_B = 16
_C = 768
_P = 512
_HW = 1024
_R = _HW // 2
_N = _B * 2
_KI = 6
_KO = 12


def _concat_pos_kernel(x_hbm, row_ref, col_ref, o_hbm, cbuf, tile, in_sems, out_sems):
    colb = jnp.broadcast_to(col_ref[...][None, :, :], (32, 32, 256)).reshape(_HW, 256)
    rowb = jnp.broadcast_to(row_ref[...][:, None, :], (32, 32, 256)).reshape(_HW, 256)
    for s in range(_KO):
        lo = (s % 2) * _R
        tile[s, :, _C:_C + 256] = colb[lo:lo + _R]
        tile[s, :, _C + 256:] = rowb[lo:lo + _R]

    def in_copy(i):
        b, half = divmod(i, 2)
        return pltpu.make_async_copy(
            x_hbm.at[b, half * _R:(half + 1) * _R],
            cbuf.at[i % _KI], in_sems.at[i % _KI])

    def out_copy(i):
        b, half = divmod(i, 2)
        return pltpu.make_async_copy(
            tile.at[i % _KO],
            o_hbm.at[b, half * _R:(half + 1) * _R], out_sems.at[i % _KO])

    out_copies = [out_copy(i) for i in range(_N)]
    for i in range(_KI):
        in_copy(i).start()
    for i in range(_N):
        in_copy(i).wait()
        if i >= _KO:
            out_copies[i - _KO].wait()
        tile[i % _KO, :, 0:_C] = cbuf[i % _KI]
        out_copies[i].start()
        if i + _KI < _N:
            in_copy(i + _KI).start()
    for i in range(_N - _KO, _N):
        out_copies[i].wait()


def kernel(x, row_embed, col_embed):
    b, c, h, w = x.shape
    xt = x.transpose(0, 2, 3, 1).reshape(b, h * w, c)
    out = pl.pallas_call(
        _concat_pos_kernel,
        in_specs=[
            pl.BlockSpec(memory_space=pl.ANY),
            pl.BlockSpec(memory_space=pltpu.MemorySpace.VMEM),
            pl.BlockSpec(memory_space=pltpu.MemorySpace.VMEM),
        ],
        out_specs=pl.BlockSpec(memory_space=pl.ANY),
        out_shape=jax.ShapeDtypeStruct((b, h * w, c + _P), x.dtype),
        scratch_shapes=[
            pltpu.VMEM((_KI, _R, c), x.dtype),
            pltpu.VMEM((_KO, _R, c + _P), x.dtype),
            pltpu.SemaphoreType.DMA((_KI,)),
            pltpu.SemaphoreType.DMA((_KO,)),
        ],
    )(xt, row_embed, col_embed)
    return out.reshape(b, h, w, c + _P).transpose(0, 3, 1, 2)


# speedup vs baseline: 1.0167x; 1.0030x over previous
import jax
import jax.numpy as jnp
from jax.experimental import pallas as pl
import jax.experimental.pallas.tpu as pltpu

_B = 16
_C = 768
_P = 512
_HW = 1024
_KI = 4
_KO = 6


def _concat_pos_kernel(x_hbm, row_ref, col_ref, o_hbm, cbuf, tile, in_sems, out_sems):
    colb = jnp.broadcast_to(col_ref[...][None, :, :], (32, 32, 256)).reshape(_HW, 256)
    rowb = jnp.broadcast_to(row_ref[...][:, None, :], (32, 32, 256)).reshape(_HW, 256)
    for s in range(_KO):
        tile[s, :, _C:_C + 256] = colb
        tile[s, :, _C + 256:] = rowb

    def in_copy(i):
        return pltpu.make_async_copy(x_hbm.at[i], cbuf.at[i % _KI], in_sems.at[i % _KI])

    out_copies = [
        pltpu.make_async_copy(tile.at[i % _KO], o_hbm.at[i], out_sems.at[i % _KO])
        for i in range(_B)
    ]

    for i in range(_KI):
        in_copy(i).start()
    for i in range(_B):
        in_copy(i).wait()
        if i >= _KO:
            out_copies[i - _KO].wait()
        tile[i % _KO, :, 0:_C] = cbuf[i % _KI]
        out_copies[i].start()
        if i + _KI < _B:
            in_copy(i + _KI).start()
    for i in range(_B - _KO, _B):
        out_copies[i].wait()


def kernel(x, row_embed, col_embed):
    b, c, h, w = x.shape
    xt = x.transpose(0, 2, 3, 1).reshape(b, h * w, c)
    out = pl.pallas_call(
        _concat_pos_kernel,
        in_specs=[
            pl.BlockSpec(memory_space=pl.ANY),
            pl.BlockSpec(memory_space=pltpu.MemorySpace.VMEM),
            pl.BlockSpec(memory_space=pltpu.MemorySpace.VMEM),
        ],
        out_specs=pl.BlockSpec(memory_space=pl.ANY),
        out_shape=jax.ShapeDtypeStruct((b, h * w, c + _P), x.dtype),
        scratch_shapes=[
            pltpu.VMEM((_KI, h * w, c), x.dtype),
            pltpu.VMEM((_KO, h * w, c + _P), x.dtype),
            pltpu.SemaphoreType.DMA((_KI,)),
            pltpu.SemaphoreType.DMA((_KO,)),
        ],
    )(xt, row_embed, col_embed)
    return out.reshape(b, h, w, c + _P).transpose(0, 3, 1, 2)
